# baseline (speedup 1.0000x reference)
import functools

import jax
import jax.numpy as jnp
from jax import lax
from jax.experimental import pallas as pl
from jax.experimental.pallas import tpu as pltpu

N = 10000
NP = 10240
E = 320000
BLK = 128


def _mm_block(x_ref, w_ref, o_ref):
    o_ref[...] = jnp.dot(x_ref[...], w_ref[...],
                         preferred_element_type=jnp.float32)


def _matmul(x, w):
    n, k = x.shape
    _, m = w.shape
    return pl.pallas_call(
        _mm_block,
        grid=(n // BLK,),
        in_specs=[pl.BlockSpec((BLK, k), lambda i: (i, 0)),
                  pl.BlockSpec((k, m), lambda i: (0, 0))],
        out_specs=pl.BlockSpec((BLK, m), lambda i: (i, 0)),
        out_shape=jax.ShapeDtypeStruct((n, m), jnp.float32),
    )(x, w)


def _post1_block(s_ref, r_ref, b_ref, w_ref, o_ref):
    s = s_ref[0] + s_ref[1]
    cnt = jnp.maximum(s[:, 64:65], 1.0)
    h = s[:, :64] / cnt + r_ref[...] + b_ref[...]
    h = jnp.where(h > 0, h, jnp.exp(jnp.minimum(h, 0.0)) - 1.0)
    o_ref[...] = jnp.dot(h, w_ref[...], preferred_element_type=jnp.float32)


def _post1(sums1, r1, b1, wc2):
    return pl.pallas_call(
        _post1_block,
        grid=(NP // BLK,),
        in_specs=[pl.BlockSpec((2, BLK, 72), lambda i: (0, i, 0)),
                  pl.BlockSpec((BLK, 64), lambda i: (i, 0)),
                  pl.BlockSpec((1, 64), lambda i: (0, 0)),
                  pl.BlockSpec((64, 96), lambda i: (0, 0))],
        out_specs=pl.BlockSpec((BLK, 96), lambda i: (i, 0)),
        out_shape=jax.ShapeDtypeStruct((NP, 96), jnp.float32),
    )(sums1, r1, b1, wc2)


def _post2_block(s2_ref, s1_ref, r_ref, b_ref, o_ref):
    t = s2_ref[0] + s2_ref[1]
    s1 = s1_ref[0] + s1_ref[1]
    cnt = jnp.maximum(s1[:, 64:65], 1.0)
    t = t / cnt + r_ref[...] + b_ref[...]
    mx = jnp.max(t, axis=1, keepdims=True)
    lse = jnp.log(jnp.sum(jnp.exp(t - mx), axis=1, keepdims=True)) + mx
    o_ref[...] = t - lse


def _post2(sums2, sums1, r2, b2):
    return pl.pallas_call(
        _post2_block,
        grid=(NP // BLK,),
        in_specs=[pl.BlockSpec((2, BLK, 32), lambda i: (0, i, 0)),
                  pl.BlockSpec((2, BLK, 72), lambda i: (0, i, 0)),
                  pl.BlockSpec((BLK, 32), lambda i: (i, 0)),
                  pl.BlockSpec((1, 32), lambda i: (0, 0))],
        out_specs=pl.BlockSpec((BLK, 32), lambda i: (i, 0)),
        out_shape=jax.ShapeDtypeStruct((NP, 32), jnp.float32),
    )(sums2, sums1, r2, b2)


def _edge_phase(ab, src, dst, u, h, with_cnt):
    a = jnp.take(ab[:, :h], src, axis=0)
    b = jnp.take(ab[:, h:2 * h], src, axis=0)
    m = a + u[:, None] * b
    w = h + 8 if with_cnt else h
    if with_cnt:
        ones = jnp.ones((m.shape[0], 1), jnp.float32)
        zeros = jnp.zeros((m.shape[0], 7), jnp.float32)
        m = jnp.concatenate([m, ones, zeros], axis=1)
    sums = jax.ops.segment_sum(m, dst, num_segments=NP)
    out = jnp.zeros((2, NP, w), jnp.float32).at[0].set(sums)
    return out


def kernel(x, edge_index, edge_attr, W1, root1, b1, W2, root2, b2):
    xp = jnp.pad(x, ((0, NP - N), (0, 0)))
    src = edge_index[0]
    dst = edge_index[1]
    u = edge_attr[:, 0]

    wc1 = jnp.concatenate([W1[0], W1[1] - W1[0], root1], axis=1)
    p1 = _matmul(xp, wc1)
    sums1 = _edge_phase(p1[:, :128], src, dst, u, 64, True)

    wc2 = jnp.concatenate([W2[0], W2[1] - W2[0], root2], axis=1)
    p2 = _post1(sums1, p1[:, 128:], b1[None, :], wc2)
    sums2 = _edge_phase(p2[:, :64], src, dst, u, 32, False)

    out = _post2(sums2, sums1, p2[:, 64:], b2[None, :])
    return out[:N]


# speedup vs baseline: 4.1714x; 4.1714x over previous
import functools

import jax
import jax.numpy as jnp
from jax import lax
from jax.experimental import pallas as pl
from jax.experimental.pallas import tpu as pltpu
from jax.experimental.pallas import tpu_sc as plsc

N = 10000
NP = 10240
E = 320000
EK = 128
EP = ((E + 32 * EK - 1) // (32 * EK)) * (32 * EK)
BLK = 128


def _mm_block(x_ref, w_ref, o_ref):
    o_ref[...] = jnp.dot(x_ref[...], w_ref[...],
                         preferred_element_type=jnp.float32)


def _matmul(x, w):
    n, k = x.shape
    _, m = w.shape
    return pl.pallas_call(
        _mm_block,
        grid=(n // BLK,),
        in_specs=[pl.BlockSpec((BLK, k), lambda i: (i, 0)),
                  pl.BlockSpec((k, m), lambda i: (0, 0))],
        out_specs=pl.BlockSpec((BLK, m), lambda i: (i, 0)),
        out_shape=jax.ShapeDtypeStruct((n, m), jnp.float32),
    )(x, w)


def _post1_block(s_ref, r_ref, b_ref, w_ref, o_ref):
    s = s_ref[0] + s_ref[1]
    cnt = jnp.maximum(s[:, 64:65], 1.0)
    h = s[:, :64] / cnt + r_ref[...] + b_ref[...]
    h = jnp.where(h > 0, h, jnp.exp(jnp.minimum(h, 0.0)) - 1.0)
    o_ref[...] = jnp.dot(h, w_ref[...], preferred_element_type=jnp.float32)


def _post1(sums1, r1, b1, wc2):
    return pl.pallas_call(
        _post1_block,
        grid=(NP // BLK,),
        in_specs=[pl.BlockSpec((2, BLK, 80), lambda i: (0, i, 0)),
                  pl.BlockSpec((BLK, 64), lambda i: (i, 0)),
                  pl.BlockSpec((1, 64), lambda i: (0, 0)),
                  pl.BlockSpec((64, 96), lambda i: (0, 0))],
        out_specs=pl.BlockSpec((BLK, 96), lambda i: (i, 0)),
        out_shape=jax.ShapeDtypeStruct((NP, 96), jnp.float32),
    )(sums1, r1, b1, wc2)


def _post2_block(s2_ref, s1_ref, r_ref, b_ref, o_ref):
    t = s2_ref[0] + s2_ref[1]
    s1 = s1_ref[0] + s1_ref[1]
    cnt = jnp.maximum(s1[:, 64:65], 1.0)
    t = t / cnt + r_ref[...] + b_ref[...]
    mx = jnp.max(t, axis=1, keepdims=True)
    lse = jnp.log(jnp.sum(jnp.exp(t - mx), axis=1, keepdims=True)) + mx
    o_ref[...] = t - lse

def _post2(sums2, sums1, r2, b2):
    return pl.pallas_call(
        _post2_block,
        grid=(NP // BLK,),
        in_specs=[pl.BlockSpec((2, BLK, 32), lambda i: (0, i, 0)),
                  pl.BlockSpec((2, BLK, 80), lambda i: (0, i, 0)),
                  pl.BlockSpec((BLK, 32), lambda i: (i, 0)),
                  pl.BlockSpec((1, 32), lambda i: (0, 0))],
        out_specs=pl.BlockSpec((BLK, 32), lambda i: (i, 0)),
        out_shape=jax.ShapeDtypeStruct((NP, 32), jnp.float32),
    )(sums2, sums1, r2, b2)


_GATHER_DNUMS = lax.GatherDimensionNumbers(
    offset_dims=(), collapsed_slice_dims=(0,), start_index_map=(0,))


def _bcast_lane(v, e):
    idx = jnp.full((16, 1), e, jnp.int32)
    return lax.gather(v, idx, _GATHER_DNUMS, slice_sizes=(1,),
                      mode=lax.GatherScatterMode.PROMISE_IN_BOUNDS)


def _make_edge_kernel(h, w):
    two_h = 2 * h
    ept = EP // 32
    n_chunks = ept // EK
    rpt = NP // 16
    mesh = plsc.VectorSubcoreMesh(core_axis_name="c", subcore_axis_name="s")

    @functools.partial(
        pl.kernel, mesh=mesh,
        compiler_params=pltpu.CompilerParams(use_tc_tiling_on_sc=False),
        out_type=jax.ShapeDtypeStruct((2, NP, w), jnp.float32),
        scratch_types=[
            pltpu.VMEM_SHARED((NP, w), jnp.float32),
            pltpu.VMEM((EK,), jnp.int32),
            pltpu.VMEM((EK,), jnp.int32),
            pltpu.VMEM((EK,), jnp.float32),
            pltpu.VMEM((EK, two_h), jnp.float32),
            pltpu.VMEM((EK, w), jnp.float32),
            pltpu.SemaphoreType.DMA,
        ])
    def edge_kernel(ab_hbm, src_hbm, dst_hbm, u_hbm, out_hbm,
                    acc_sh, src_v, dst_v, u_v, rows_v, m_v, sem):
        c = lax.axis_index("c")
        s = lax.axis_index("s")
        zero16 = jnp.zeros((16,), jnp.float32)

        def zrow(e, carry):
            for kk in range(w // 16):
                m_v[e, pl.ds(kk * 16, 16)] = zero16
            return carry
        lax.fori_loop(0, EK, zrow, 0)
        for r in range(rpt // EK):
            pltpu.sync_copy(m_v, acc_sh.at[pl.ds(s * rpt + r * EK, EK)])
        if w > h:
            ii = lax.iota(jnp.int32, 16)
            onehot = (1 - jnp.minimum(ii, 1)).astype(jnp.float32)
            def crow(e, carry):
                m_v[e, pl.ds(h, 16)] = onehot
                return carry
            lax.fori_loop(0, EK, crow, 0)
        plsc.subcore_barrier()

        base0 = (c * 16 + s) * ept

        def chunk(i, carry):
            base = base0 + i * EK
            pltpu.sync_copy(src_hbm.at[pl.ds(base, EK)], src_v)
            pltpu.sync_copy(dst_hbm.at[pl.ds(base, EK)], dst_v)
            pltpu.sync_copy(u_hbm.at[pl.ds(base, EK)], u_v)
            pltpu.async_copy(ab_hbm.at[src_v], rows_v, sem).wait()

            def grp(g, carry2):
                uvec = u_v[pl.ds(g * 16, 16)]
                for e in range(16):
                    ue = _bcast_lane(uvec, e)
                    row = g * 16 + e
                    for kk in range(h // 16):
                        a = rows_v[row, pl.ds(kk * 16, 16)]
                        b = rows_v[row, pl.ds(h + kk * 16, 16)]
                        m_v[row, pl.ds(kk * 16, 16)] = a + ue * b
                return carry2
            lax.fori_loop(0, EK // 16, grp, 0)

            pltpu.sync_copy(m_v, acc_sh.at[dst_v], add=True)
            return carry
        lax.fori_loop(0, n_chunks, chunk, 0)

        plsc.subcore_barrier()
        pltpu.sync_copy(acc_sh.at[pl.ds(s * rpt, rpt)],
                        out_hbm.at[c, pl.ds(s * rpt, rpt)])

    return edge_kernel


_edge_l1 = _make_edge_kernel(64, 80)
_edge_l2 = _make_edge_kernel(32, 32)


def kernel(x, edge_index, edge_attr, W1, root1, b1, W2, root2, b2):
    xp = jnp.pad(x, ((0, NP - N), (0, 0)))
    pad = EP - E
    src = jnp.concatenate([edge_index[0], jnp.full((pad,), N, jnp.int32)])
    dst = jnp.concatenate([edge_index[1], jnp.full((pad,), N, jnp.int32)])
    u = jnp.concatenate([edge_attr[:, 0], jnp.zeros((pad,), jnp.float32)])

    wc1 = jnp.concatenate([W1[0], W1[1] - W1[0], root1], axis=1)
    p1 = _matmul(xp, wc1)
    sums1 = _edge_l1(p1[:, :128], src, dst, u)

    wc2 = jnp.concatenate([W2[0], W2[1] - W2[0], root2], axis=1)
    p2 = _post1(sums1, p1[:, 128:], b1[None, :], wc2)
    sums2 = _edge_l2(p2[:, :64], src, dst, u)

    out = _post2(sums2, sums1, p2[:, 64:], b2[None, :])
    return out[:N]


# speedup vs baseline: 5.3769x; 1.2890x over previous
import functools

import jax
import jax.numpy as jnp
from jax import lax
from jax.experimental import pallas as pl
from jax.experimental.pallas import tpu as pltpu
from jax.experimental.pallas import tpu_sc as plsc

N = 10000
NP = 10240
E = 320000
EK = 128
NCH = 80
EP = 32 * EK * NCH
BLK = 128


def _mm_block(x_ref, w_ref, o_ref):
    o_ref[...] = jnp.dot(x_ref[...], w_ref[...],
                         preferred_element_type=jnp.float32)


def _matmul(x, w):
    n, k = x.shape
    _, m = w.shape
    return pl.pallas_call(
        _mm_block,
        grid=(n // BLK,),
        in_specs=[pl.BlockSpec((BLK, k), lambda i: (i, 0)),
                  pl.BlockSpec((k, m), lambda i: (0, 0))],
        out_specs=pl.BlockSpec((BLK, m), lambda i: (i, 0)),
        out_shape=jax.ShapeDtypeStruct((n, m), jnp.float32),
    )(x, w)


def _post1_block(s_ref, r_ref, b_ref, w_ref, o_ref):
    s = s_ref[0] + s_ref[1]
    cnt = jnp.maximum(s[:, 64:65], 1.0)
    h = s[:, :64] / cnt + r_ref[...] + b_ref[...]
    h = jnp.where(h > 0, h, jnp.exp(jnp.minimum(h, 0.0)) - 1.0)
    o_ref[...] = jnp.dot(h, w_ref[...], preferred_element_type=jnp.float32)


def _post1(sums1, r1, b1, wc2):
    return pl.pallas_call(
        _post1_block,
        grid=(NP // BLK,),
        in_specs=[pl.BlockSpec((2, BLK, 80), lambda i: (0, i, 0)),
                  pl.BlockSpec((BLK, 64), lambda i: (i, 0)),
                  pl.BlockSpec((1, 64), lambda i: (0, 0)),
                  pl.BlockSpec((64, 96), lambda i: (0, 0))],
        out_specs=pl.BlockSpec((BLK, 96), lambda i: (i, 0)),
        out_shape=jax.ShapeDtypeStruct((NP, 96), jnp.float32),
    )(sums1, r1, b1, wc2)


def _post2_block(s2_ref, s1_ref, r_ref, b_ref, o_ref):
    t = s2_ref[0] + s2_ref[1]
    s1 = s1_ref[0] + s1_ref[1]
    cnt = jnp.maximum(s1[:, 64:65], 1.0)
    t = t / cnt + r_ref[...] + b_ref[...]
    mx = jnp.max(t, axis=1, keepdims=True)
    lse = jnp.log(jnp.sum(jnp.exp(t - mx), axis=1, keepdims=True)) + mx
    o_ref[...] = t - lse


def _post2(sums2, sums1, r2, b2):
    return pl.pallas_call(
        _post2_block,
        grid=(NP // BLK,),
        in_specs=[pl.BlockSpec((2, BLK, 32), lambda i: (0, i, 0)),
                  pl.BlockSpec((2, BLK, 80), lambda i: (0, i, 0)),
                  pl.BlockSpec((BLK, 32), lambda i: (i, 0)),
                  pl.BlockSpec((1, 32), lambda i: (0, 0))],
        out_specs=pl.BlockSpec((BLK, 32), lambda i: (i, 0)),
        out_shape=jax.ShapeDtypeStruct((NP, 32), jnp.float32),
    )(sums2, sums1, r2, b2)


_GATHER_DNUMS = lax.GatherDimensionNumbers(
    offset_dims=(), collapsed_slice_dims=(0,), start_index_map=(0,))


def _bcast_lane(v, e):
    idx = jnp.full((16, 1), e, jnp.int32)
    return lax.gather(v, idx, _GATHER_DNUMS, slice_sizes=(1,),
                      mode=lax.GatherScatterMode.PROMISE_IN_BOUNDS)


def _make_edge_kernel(h, w):
    two_h = 2 * h
    rpt = NP // 16
    nchh = NCH // 2
    mesh = plsc.VectorSubcoreMesh(core_axis_name="c", subcore_axis_name="s")

    @functools.partial(
        pl.kernel, mesh=mesh,
        compiler_params=pltpu.CompilerParams(use_tc_tiling_on_sc=False),
        out_type=jax.ShapeDtypeStruct((2, NP, w), jnp.float32),
        scratch_types=[
            pltpu.VMEM_SHARED((NP, w), jnp.float32),
            pltpu.VMEM((nchh, EK), jnp.int32),
            pltpu.VMEM((nchh, EK), jnp.int32),
            pltpu.VMEM((nchh, EK), jnp.float32),
            pltpu.VMEM((2, EK, two_h), jnp.float32),
            pltpu.VMEM((2, EK, w), jnp.float32),
            pltpu.SemaphoreType.DMA,
            pltpu.SemaphoreType.DMA,
            pltpu.SemaphoreType.DMA,
            pltpu.SemaphoreType.DMA,
        ])
    def edge_kernel(ab_hbm, src_hbm, dst_hbm, u_hbm, out_hbm,
                    acc_sh, srcb, dstb, ub, rows, mb,
                    gsem0, gsem1, ssem0, ssem1):
        cid = lax.axis_index("c")
        sid = lax.axis_index("s")
        tid = cid * 16 + sid
        gsems = (gsem0, gsem1)
        ssems = (ssem0, ssem1)
        zero16 = jnp.zeros((16,), jnp.float32)

        def zrow(e, carry):
            for kk in range(w // 16):
                mb[0, e, pl.ds(kk * 16, 16)] = zero16
            return carry
        lax.fori_loop(0, EK, zrow, 0)
        for r in range(rpt // EK):
            pltpu.sync_copy(mb.at[0], acc_sh.at[pl.ds(sid * rpt + r * EK, EK)])
        if w > h:
            ii = lax.iota(jnp.int32, 16)
            onehot = (1 - jnp.minimum(ii, 1)).astype(jnp.float32)
            def crow(e, carry):
                mb[0, e, pl.ds(h, 16)] = onehot
                mb[1, e, pl.ds(h, 16)] = onehot
                return carry
            lax.fori_loop(0, EK, crow, 0)
        plsc.subcore_barrier()

        for half in range(2):
            off = tid * NCH + half * nchh
            pltpu.sync_copy(src_hbm.at[pl.ds(off, nchh)], srcb)
            pltpu.sync_copy(dst_hbm.at[pl.ds(off, nchh)], dstb)
            pltpu.sync_copy(u_hbm.at[pl.ds(off, nchh)], ub)

            for b in range(2):
                pltpu.async_copy(ab_hbm.at[srcb.at[b]], rows.at[b], gsems[b])

            def pair(i, carry):
                for b in range(2):
                    ch = i * 2 + b
                    pltpu.make_async_copy(
                        ab_hbm.at[srcb.at[ch]], rows.at[b], gsems[b]).wait()
                    @pl.when(ch >= 2)
                    def _():
                        pltpu.make_async_copy(
                            mb.at[b], acc_sh.at[dstb.at[ch - 2]],
                            ssems[b]).wait()

                    def grp(g, carry2):
                        uvec = ub[ch, pl.ds(g * 16, 16)]
                        for e in range(16):
                            ue = _bcast_lane(uvec, e)
                            row = g * 16 + e
                            for kk in range(h // 16):
                                a = rows[b, row, pl.ds(kk * 16, 16)]
                                bb = rows[b, row, pl.ds(h + kk * 16, 16)]
                                mb[b, row, pl.ds(kk * 16, 16)] = a + ue * bb
                        return carry2
                    lax.fori_loop(0, EK // 16, grp, 0)

                    @pl.when(ch + 2 < nchh)
                    def _():
                        pltpu.async_copy(
                            ab_hbm.at[srcb.at[ch + 2]], rows.at[b], gsems[b])
                    pltpu.async_copy(
                        mb.at[b], acc_sh.at[dstb.at[ch]], ssems[b], add=True)
                return carry
            lax.fori_loop(0, nchh // 2, pair, 0)

            for b in range(2):
                pltpu.make_async_copy(
                    mb.at[b], acc_sh.at[dstb.at[nchh - 2 + b]],
                    ssems[b]).wait()

        plsc.subcore_barrier()
        pltpu.sync_copy(acc_sh.at[pl.ds(sid * rpt, rpt)],
                        out_hbm.at[cid, pl.ds(sid * rpt, rpt)])

    return edge_kernel


_edge_l1 = _make_edge_kernel(64, 80)
_edge_l2 = _make_edge_kernel(32, 32)


def kernel(x, edge_index, edge_attr, W1, root1, b1, W2, root2, b2):
    xp = jnp.pad(x, ((0, NP - N), (0, 0)))
    pad = EP - E
    src = jnp.concatenate([edge_index[0], jnp.full((pad,), N, jnp.int32)])
    dst = jnp.concatenate([edge_index[1], jnp.full((pad,), N, jnp.int32)])
    u = jnp.concatenate([edge_attr[:, 0], jnp.zeros((pad,), jnp.float32)])
    src = src.reshape(EP // EK, EK)
    dst = dst.reshape(EP // EK, EK)
    u = u.reshape(EP // EK, EK)

    wc1 = jnp.concatenate([W1[0], W1[1] - W1[0], root1], axis=1)
    p1 = _matmul(xp, wc1)
    sums1 = _edge_l1(p1[:, :128], src, dst, u)

    wc2 = jnp.concatenate([W2[0], W2[1] - W2[0], root2], axis=1)
    p2 = _post1(sums1, p1[:, 128:], b1[None, :], wc2)
    sums2 = _edge_l2(p2[:, :64], src, dst, u)

    out = _post2(sums2, sums1, p2[:, 64:], b2[None, :])
    return out[:N]


# speedup vs baseline: 12.0841x; 2.2474x over previous
import functools

import jax
import jax.numpy as jnp
import numpy as np
from jax import lax
from jax.experimental import pallas as pl
from jax.experimental.pallas import tpu as pltpu
from jax.experimental.pallas import tpu_sc as plsc

N = 10000
NP = 10240
E = 320000
EK = 64
ND = 2
NCH = 160
NCHR = 40
EP = 32 * EK * NCH
BLK = 128


def _mm1_block(x_ref, wab_ref, wr_ref, ab_ref, r_ref):
    xb = x_ref[...]
    ab = jnp.dot(xb, wab_ref[...], preferred_element_type=jnp.float32)
    ab_ref[...] = ab.astype(jnp.bfloat16)
    r_ref[...] = jnp.dot(xb, wr_ref[...], preferred_element_type=jnp.float32)


def _mm1(x, wab, wr):
    return pl.pallas_call(
        _mm1_block,
        grid=(NP // BLK,),
        in_specs=[pl.BlockSpec((BLK, 128), lambda i: (i, 0)),
                  pl.BlockSpec((128, 128), lambda i: (0, 0)),
                  pl.BlockSpec((128, 64), lambda i: (0, 0))],
        out_specs=[pl.BlockSpec((BLK, 128), lambda i: (i, 0)),
                   pl.BlockSpec((BLK, 64), lambda i: (i, 0))],
        out_shape=[jax.ShapeDtypeStruct((NP, 128), jnp.bfloat16),
                   jax.ShapeDtypeStruct((NP, 64), jnp.float32)],
    )(x, wab, wr)


def _post1_block(s_ref, r_ref, b_ref, w_ref, ab_ref, r2_ref):
    s = s_ref[0] + s_ref[1]
    cnt = jnp.maximum(s[:, 64:65], 1.0)
    h = s[:, :64] / cnt + r_ref[...] + b_ref[...]
    h = jnp.where(h > 0, h, jnp.exp(jnp.minimum(h, 0.0)) - 1.0)
    p2 = jnp.dot(h, w_ref[...], preferred_element_type=jnp.float32)
    ab_ref[...] = p2[:, :64]
    r2_ref[...] = p2[:, 64:]


def _post1(sums1, r1, b1, wc2):
    return pl.pallas_call(
        _post1_block,
        grid=(NP // BLK,),
        in_specs=[pl.BlockSpec((2, BLK, 80), lambda i: (0, i, 0)),
                  pl.BlockSpec((BLK, 64), lambda i: (i, 0)),
                  pl.BlockSpec((1, 64), lambda i: (0, 0)),
                  pl.BlockSpec((64, 96), lambda i: (0, 0))],
        out_specs=[pl.BlockSpec((BLK, 64), lambda i: (i, 0)),
                   pl.BlockSpec((BLK, 32), lambda i: (i, 0))],
        out_shape=[jax.ShapeDtypeStruct((NP, 64), jnp.float32),
                   jax.ShapeDtypeStruct((NP, 32), jnp.float32)],
    )(sums1, r1, b1, wc2)


def _post2_block(s2_ref, s1_ref, r_ref, b_ref, o_ref):
    t = s2_ref[0] + s2_ref[1]
    s1 = s1_ref[0] + s1_ref[1]
    cnt = jnp.maximum(s1[:, 64:65], 1.0)
    t = t / cnt + r_ref[...] + b_ref[...]
    mx = jnp.max(t, axis=1, keepdims=True)
    lse = jnp.log(jnp.sum(jnp.exp(t - mx), axis=1, keepdims=True)) + mx
    o_ref[...] = t - lse


def _post2(sums2, sums1, r2, b2):
    return pl.pallas_call(
        _post2_block,
        grid=(NP // BLK,),
        in_specs=[pl.BlockSpec((2, BLK, 32), lambda i: (0, i, 0)),
                  pl.BlockSpec((2, BLK, 80), lambda i: (0, i, 0)),
                  pl.BlockSpec((BLK, 32), lambda i: (i, 0)),
                  pl.BlockSpec((1, 32), lambda i: (0, 0))],
        out_specs=pl.BlockSpec((BLK, 32), lambda i: (i, 0)),
        out_shape=jax.ShapeDtypeStruct((NP, 32), jnp.float32),
    )(sums2, sums1, r2, b2)


_GATHER_DNUMS = lax.GatherDimensionNumbers(
    offset_dims=(), collapsed_slice_dims=(0,), start_index_map=(0,))


def _bcast_lane(v, e):
    idx = jnp.full((16, 1), e, jnp.int32)
    return lax.gather(v, idx, _GATHER_DNUMS, slice_sizes=(1,),
                      mode=lax.GatherScatterMode.PROMISE_IN_BOUNDS)


def _make_edge_kernel(h, w, packed):
    two_h = 2 * h
    tdt = jnp.bfloat16 if packed else jnp.float32
    rpt = NP // 16
    rounds = NCH // NCHR
    nk = h // 16
    mesh = plsc.VectorSubcoreMesh(core_axis_name="c", subcore_axis_name="s")

    @functools.partial(
        pl.kernel, mesh=mesh,
        compiler_params=pltpu.CompilerParams(use_tc_tiling_on_sc=False,
                                             needs_layout_passes=False),
        out_type=jax.ShapeDtypeStruct((2, NP, w), jnp.float32),
        scratch_types=[
            pltpu.VMEM_SHARED((NP, two_h), tdt),
            pltpu.VMEM_SHARED((NP, w), jnp.float32),
            pltpu.VMEM((NCHR, EK), jnp.int32),
            pltpu.VMEM((NCHR, EK), jnp.int32),
            pltpu.VMEM((NCHR, EK), jnp.float32),
            pltpu.VMEM((ND, EK, two_h), tdt),
            pltpu.VMEM((ND, EK, w), jnp.float32),
        ] + [pltpu.SemaphoreType.DMA] * (2 * ND))
    def edge_kernel(ab_hbm, src_hbm, dst_hbm, u_hbm, out_hbm,
                    ab_sh, acc_sh, srcb, dstb, ub, rows, mb, *sems):
        cid = lax.axis_index("c")
        sid = lax.axis_index("s")
        tid = cid * 16 + sid
        gsems = sems[:ND]
        ssems = sems[ND:]
        zero16 = jnp.zeros((16,), jnp.float32)

        pltpu.sync_copy(ab_hbm.at[pl.ds(sid * rpt, rpt)],
                        ab_sh.at[pl.ds(sid * rpt, rpt)])

        def zrow(e, carry):
            for d in range(ND):
                for kk in range(w // 16):
                    mb[d, e, pl.ds(kk * 16, 16)] = zero16
            return carry
        lax.fori_loop(0, EK, zrow, 0)
        for r in range(rpt // EK):
            pltpu.sync_copy(
                mb.at[0], acc_sh.at[pl.ds(sid * rpt + r * EK, EK)])
        if w > h:
            ii = lax.iota(jnp.int32, 16)
            onehot = (1 - jnp.minimum(ii, 1)).astype(jnp.float32)
            def crow(e, carry):
                for d in range(ND):
                    mb[d, e, pl.ds(h, 16)] = onehot
                return carry
            lax.fori_loop(0, EK, crow, 0)
        plsc.subcore_barrier()

        def round_body(rd, carry0):
            off = tid * NCH + rd * NCHR
            pltpu.sync_copy(src_hbm.at[pl.ds(off, NCHR)], srcb)
            pltpu.sync_copy(dst_hbm.at[pl.ds(off, NCHR)], dstb)
            pltpu.sync_copy(u_hbm.at[pl.ds(off, NCHR)], ub)

            for b in range(ND):
                pltpu.async_copy(ab_sh.at[srcb.at[b]], rows.at[b], gsems[b])

            def pair(i, carry):
                for b in range(ND):
                    ch = i * ND + b
                    pltpu.make_async_copy(
                        ab_sh.at[srcb.at[ch]], rows.at[b], gsems[b]).wait()
                    @pl.when(ch >= ND)
                    def _():
                        pltpu.make_async_copy(
                            mb.at[b], acc_sh.at[dstb.at[ch - ND]],
                            ssems[b]).wait()

                    def grp(g, carry2):
                        uvec = ub[ch, pl.ds(g * 16, 16)]
                        for e0 in range(0, 16, 4):
                            ues = [_bcast_lane(uvec, e0 + d) for d in range(4)]
                            ms = []
                            if packed:
                                for d in range(4):
                                    row = g * 16 + e0 + d
                                    blks = []
                                    for j in range(nk):
                                        lj = rows[b, row, pl.ds(j * 32, 32)]
                                        p, q = plsc.unpack(
                                            lj,
                                            format=plsc.PackFormat.INTERLEAVED)
                                        blks += [p, q]
                                    ms += [blks[kk] + ues[d] * blks[nk + kk]
                                           for kk in range(nk)]
                            else:
                                avs = [rows[b, g * 16 + e0 + d,
                                            pl.ds(kk * 16, 16)]
                                       for d in range(4) for kk in range(nk)]
                                bvs = [rows[b, g * 16 + e0 + d,
                                            pl.ds(h + kk * 16, 16)]
                                       for d in range(4) for kk in range(nk)]
                                ms = [avs[d * nk + kk]
                                      + ues[d] * bvs[d * nk + kk]
                                      for d in range(4) for kk in range(nk)]
                            for d in range(4):
                                for kk in range(nk):
                                    mb[b, g * 16 + e0 + d,
                                       pl.ds(kk * 16, 16)] = ms[d * nk + kk]
                        return carry2
                    lax.fori_loop(0, EK // 16, grp, 0)

                    @pl.when(ch + ND < NCHR)
                    def _():
                        pltpu.async_copy(
                            ab_sh.at[srcb.at[ch + ND]], rows.at[b], gsems[b])
                    pltpu.async_copy(
                        mb.at[b], acc_sh.at[dstb.at[ch]], ssems[b], add=True)
                return carry
            lax.fori_loop(0, NCHR // ND, pair, 0)

            for b in range(ND):
                pltpu.make_async_copy(
                    mb.at[b], acc_sh.at[dstb.at[NCHR - ND + b]],
                    ssems[b]).wait()
            return carry0
        lax.fori_loop(0, rounds, round_body, 0)

        plsc.subcore_barrier()
        pltpu.sync_copy(acc_sh.at[pl.ds(sid * rpt, rpt)],
                        out_hbm.at[cid, pl.ds(sid * rpt, rpt)])

    return edge_kernel


_edge_l1 = _make_edge_kernel(64, 80, True)
_edge_l2 = _make_edge_kernel(32, 32, False)


def _pair_perm(width):
    perm = np.zeros((width,), np.int32)
    for j in range(width // 32):
        for i in range(16):
            perm[32 * j + 2 * i] = 32 * j + i
            perm[32 * j + 2 * i + 1] = 32 * j + 16 + i
    return jnp.asarray(perm)


def kernel(x, edge_index, edge_attr, W1, root1, b1, W2, root2, b2):
    xp = jnp.pad(x, ((0, NP - N), (0, 0)))
    pad = EP - E
    src = jnp.concatenate([edge_index[0], jnp.full((pad,), N, jnp.int32)])
    dst = jnp.concatenate([edge_index[1], jnp.full((pad,), N, jnp.int32)])
    u = jnp.concatenate([edge_attr[:, 0], jnp.zeros((pad,), jnp.float32)])
    src = src.reshape(EP // EK, EK)
    dst = dst.reshape(EP // EK, EK)
    u = u.reshape(EP // EK, EK)

    wab1 = jnp.concatenate([W1[0], W1[1] - W1[0]], axis=1)
    wab1 = jnp.take(wab1, _pair_perm(128), axis=1)
    ab1, r1 = _mm1(xp, wab1, root1)
    sums1 = _edge_l1(ab1, src, dst, u)

    wc2 = jnp.concatenate([W2[0], W2[1] - W2[0], root2], axis=1)
    ab2, r2 = _post1(sums1, r1, b1[None, :], wc2)
    sums2 = _edge_l2(ab2, src, dst, u)

    out = _post2(sums2, sums1, r2, b2[None, :])
    return out[:N]
